# baseline (speedup 1.0000x reference)
import functools

import jax
import jax.numpy as jnp
from jax import lax
from jax.experimental import pallas as pl
from jax.experimental.pallas import tpu as pltpu
from jax.experimental.pallas import tpu_sc as plsc

_INFO = plsc.get_sparse_core_info()
_NC, _NS = _INFO.num_cores, _INFO.num_subcores
_NW = _NC * _NS

_TOK = 4


@functools.cache
def _build(T, batch, v, d):
    ncb = d // 128
    nseg = ncb * batch
    n_active = T // _TOK
    mesh = plsc.VectorSubcoreMesh(core_axis_name="c", subcore_axis_name="s")

    @functools.partial(
        pl.kernel,
        out_type=jax.ShapeDtypeStruct((T * nseg, 128), jnp.float32),
        mesh=mesh,
        scratch_types=[
            pltpu.VMEM((batch, 128), jnp.int32),
            pltpu.VMEM((_TOK * nseg,), jnp.int32),
            pltpu.VMEM((_TOK * nseg, 128), jnp.float32),
            pltpu.SemaphoreType.DMA,
        ],
    )
    def gather_kernel(ind_hbm, table_hbm, out_hbm, ind_v, idx_v, segs_v, sem):
        wid = lax.axis_index("s") * _NC + lax.axis_index("c")

        @pl.when(wid < n_active)
        def _():
            t0 = wid * _TOK
            for bi in range(batch):
                pltpu.sync_copy(ind_hbm.at[bi], ind_v.at[bi, pl.ds(0, T)])
            lane = lax.iota(jnp.int32, 16)
            b = lane & (batch - 1)
            t0a = (t0 // 16) * 16
            p = t0 - t0a
            wins = [ind_v[bi, pl.ds(t0a, 16)] for bi in range(batch)]
            dnums = lax.GatherDimensionNumbers(
                offset_dims=(), collapsed_slice_dims=(0,), start_index_map=(0,)
            )
            take = lambda w, sel: lax.gather(
                w, sel[:, None], dnums, (1,),
                mode=lax.GatherScatterMode.PROMISE_IN_BOUNDS,
            )
            for t in range(_TOK):
                sel = jnp.full((16,), p + t, jnp.int32)
                patt = take(wins[0], sel)
                for bi in range(1, batch):
                    patt = jnp.where(b == bi, take(wins[bi], sel), patt)
                for ch in range(nseg // 16):
                    cb = (ch * 16 + lane) >> 2
                    idx_v[pl.ds(t * nseg + ch * 16, 16)] = patt * ncb + cb
            copies = [
                pltpu.async_copy(
                    table_hbm.at[idx_v.at[pl.ds(k * nseg, nseg)]],
                    segs_v.at[pl.ds(k * nseg, nseg)],
                    sem,
                )
                for k in range(_TOK)
            ]
            for c in copies:
                c.wait()
            pltpu.sync_copy(segs_v, out_hbm.at[pl.ds(t0 * nseg, _TOK * nseg)])

    return gather_kernel


def kernel(indices, embedding):
    batch, t = indices.shape
    v, d = embedding.shape
    ncb = d // 128
    table2 = embedding.reshape(v * ncb, 128)
    out = _build(t, batch, v, d)(indices.astype(jnp.int32), table2)
    return (
        out.reshape(t, ncb, batch, 128).transpose(2, 0, 1, 3).reshape(batch, t, d)
    )


# speedup vs baseline: 1.0859x; 1.0859x over previous
import functools

import jax
import jax.numpy as jnp
from jax import lax
from jax.experimental import pallas as pl
from jax.experimental.pallas import tpu as pltpu
from jax.experimental.pallas import tpu_sc as plsc

_INFO = plsc.get_sparse_core_info()
_NC, _NS = _INFO.num_cores, _INFO.num_subcores
_NW = _NC * _NS

_TOK = 4


@functools.cache
def _build(T, batch, v, d):
    rows = _TOK * batch
    n_active = T // _TOK
    mesh = plsc.VectorSubcoreMesh(core_axis_name="c", subcore_axis_name="s")

    @functools.partial(
        pl.kernel,
        out_type=jax.ShapeDtypeStruct((T, batch, d), jnp.float32),
        mesh=mesh,
        scratch_types=[
            pltpu.VMEM((rows,), jnp.int32),
            pltpu.VMEM((rows, d), jnp.float32),
            pltpu.SemaphoreType.DMA,
            pltpu.SemaphoreType.DMA,
        ],
    )
    def gather_kernel(iv_hbm, table_hbm, out_hbm, idx_v, rows_v, sem, sem_w):
        wid = lax.axis_index("s") * _NC + lax.axis_index("c")

        @pl.when(wid < n_active)
        def _():
            t0 = wid * _TOK
            pltpu.sync_copy(iv_hbm.at[pl.ds(t0 * batch, rows)], idx_v)
            pltpu.async_copy(table_hbm.at[idx_v], rows_v, sem).wait()
            writes = [
                pltpu.async_copy(
                    rows_v.at[pl.ds(k * batch, batch)],
                    out_hbm.at[t0 + k],
                    sem_w,
                )
                for k in range(_TOK)
            ]
            for w in writes:
                w.wait()

    return gather_kernel


def kernel(indices, embedding):
    batch, t = indices.shape
    v, d = embedding.shape
    iv = indices.astype(jnp.int32).T.reshape(t * batch)
    out = _build(t, batch, v, d)(iv, embedding)
    return out.transpose(1, 0, 2)
